# baseline (speedup 1.0000x reference)
import functools

import jax
import jax.numpy as jnp
from jax import lax
from jax.experimental import pallas as pl
from jax.experimental.pallas import tpu as pltpu
from jax.experimental.pallas import tpu_sc as plsc

_IOU_T = 0.5
_BW, _OW, _CW = 3.54, 64.3, 37.4


def _softplus(x):
    return jnp.maximum(x, 0.0) + jnp.log1p(jnp.exp(-jnp.abs(x)))


def _assign_body(na, ns, H_list, tgt_ref, anch_ref, strd_ref,
                 mf_ref, first_ref, grow_ref, r_ref, tb_ref, tcl_ref):
    B, N = tgt_ref.shape[1], tgt_ref.shape[2]
    x = tgt_ref[0]
    y = tgt_ref[1]
    w = tgt_ref[2]
    h = tgt_ref[3]
    c = tgt_ref[4]
    valid = ~((c == -1.0) & (x == -1.0) & (y == -1.0) & (w == -1.0) & (h == -1.0))
    bidx = lax.broadcasted_iota(jnp.int32, (B, N), 0)
    tcl_ref[...] = c.astype(jnp.int32)
    for s in range(ns):
        inv = 1.0 / strd_ref[s]
        tx = x * inv
        ty = y * inv
        tw = w * inv
        th = h * inv
        cx = jnp.floor(tx)
        cy = jnp.floor(ty)
        fx = tx - cx
        fy = ty - cy
        ci = cx.astype(jnp.int32)
        cj = cy.astype(jnp.int32)
        tb_ref[s, 0] = fx
        tb_ref[s, 1] = fy
        tb_ref[s, 2] = tw
        tb_ref[s, 3] = th
        tx0 = fx - tw * 0.5
        ty0 = fy - th * 0.5
        tx1 = fx + tw * 0.5
        ty1 = fy + th * 0.5
        ta = tw * th
        Hs = H_list[s]
        for a in range(na):
            aw = anch_ref[s, a, 0] * inv
            ah = anch_ref[s, a, 1] * inv
            ax0 = 0.5 - aw * 0.5
            ay0 = 0.5 - ah * 0.5
            ax1 = 0.5 + aw * 0.5
            ay1 = 0.5 + ah * 0.5
            x0 = jnp.maximum(tx0, ax0)
            y0 = jnp.maximum(ty0, ay0)
            x1 = jnp.minimum(tx1, ax1)
            y1 = jnp.minimum(ty1, ay1)
            m = ((x0 < x1) & (y0 < y1)).astype(jnp.float32)
            inter = (x1 - x0) * (y1 - y0) * m
            iou = inter / (ta + aw * ah - inter)
            mf = (iou > _IOU_T) & valid
            row = ((bidx * na + a) * Hs + cj) * Hs + ci
            eq = row[:, :, None] == row[:, None, :]
            mfj = mf[:, None, :]
            ii = lax.broadcasted_iota(jnp.int32, (B, N, N), 1)
            jj = lax.broadcasted_iota(jnp.int32, (B, N, N), 2)
            dup = jnp.any(eq & mfj & (jj < ii), axis=2)
            sa = s * na + a
            mf_ref[sa] = mf.astype(jnp.float32)
            first_ref[sa] = (mf & ~dup).astype(jnp.float32)
            rowm = jnp.where(mf, row, 0)
            elem = rowm * 85
            grow_ref[sa] = lax.shift_right_logical(elem, 7)
            r_ref[sa] = lax.bitwise_and(elem, 127)


def _run_assign(tgtT, anchors, strides, na, ns, H_list):
    B, N = tgtT.shape[1], tgtT.shape[2]
    body = functools.partial(_assign_body, na, ns, H_list)
    return pl.pallas_call(
        body,
        in_specs=[
            pl.BlockSpec(memory_space=pltpu.VMEM),
            pl.BlockSpec(memory_space=pltpu.SMEM),
            pl.BlockSpec(memory_space=pltpu.SMEM),
        ],
        out_specs=[pl.BlockSpec(memory_space=pltpu.VMEM)] * 6,
        out_shape=[
            jax.ShapeDtypeStruct((ns * na, B, N), jnp.float32),
            jax.ShapeDtypeStruct((ns * na, B, N), jnp.float32),
            jax.ShapeDtypeStruct((ns * na, B, N), jnp.int32),
            jax.ShapeDtypeStruct((ns * na, B, N), jnp.int32),
            jax.ShapeDtypeStruct((ns, 4, B, N), jnp.float32),
            jax.ShapeDtypeStruct((B, N), jnp.int32),
        ],
    )(tgtT, anchors, strides)


def _sc_gather3(tables, idxs, rows_per_worker, W):
    NROW = idxs[0].shape[0]
    mesh = plsc.VectorSubcoreMesh(core_axis_name="c", subcore_axis_name="s")
    info = plsc.get_sparse_core_info()
    ncore = info.num_cores
    half = rows_per_worker // 2

    @functools.partial(
        pl.kernel,
        out_type=[jax.ShapeDtypeStruct((NROW, W), jnp.float32)] * 3,
        mesh=mesh,
        compiler_params=pltpu.CompilerParams(use_tc_tiling_on_sc=False),
        scratch_types=[
            [pltpu.VMEM((rows_per_worker,), jnp.int32)] * 3,
            [pltpu.VMEM((rows_per_worker, W), jnp.float32)] * 3,
            pltpu.SemaphoreType.DMA,
        ],
    )
    def gather_k(t0, t1, t2, i0, i1, i2, o0, o1, o2, idx_v, rows_v, sem):
        wid = lax.axis_index("s") * ncore + lax.axis_index("c")
        base = wid * rows_per_worker
        tabs, idx_hbm, outs = (t0, t1, t2), (i0, i1, i2), (o0, o1, o2)
        for s in range(3):
            pltpu.sync_copy(idx_hbm[s].at[pl.ds(base, rows_per_worker)],
                            idx_v[s])
        copies = []
        for s in range(3):
            for h in range(2):
                copies.append(pltpu.async_copy(
                    tabs[s].at[idx_v[s].at[pl.ds(h * half, half)]],
                    rows_v[s].at[pl.ds(h * half, half)], sem))
        for cp in copies:
            cp.wait()
        for s in range(3):
            pltpu.sync_copy(rows_v[s], outs[s].at[pl.ds(base, rows_per_worker)])

    return gather_k(*tables, *idxs)


def _obj_body(C, x_ref, o_ref):
    i = pl.program_id(0)

    @pl.when(i == 0)
    def _():
        o_ref[...] = jnp.zeros_like(o_ref)

    W = 16 * C
    col = lax.broadcasted_iota(jnp.int32, (W, 16), 0)
    jj = lax.broadcasted_iota(jnp.int32, (W, 16), 1)
    sel = (col == C * jj + 4).astype(jnp.float32)
    x4 = jax.lax.dot_general(x_ref[...], sel, (((1,), (0,)), ((), ())),
                             preferred_element_type=jnp.float32)
    o_ref[...] += jnp.sum(_softplus(x4)).reshape(1, 1)


def _run_objsum(flat16rows, C):
    G, W = flat16rows.shape
    block_rows = min(1024, G)
    grid = G // block_rows
    return pl.pallas_call(
        functools.partial(_obj_body, C),
        grid=(grid,),
        in_specs=[pl.BlockSpec((block_rows, W), lambda i: (i, 0))],
        out_specs=pl.BlockSpec((1, 1), lambda i: (0, 0)),
        out_shape=jax.ShapeDtypeStruct((1, 1), jnp.float32),
    )(flat16rows)


def _combine_body(ns, nc, M_list, g_refs, mf_ref, first_ref, r_ref, tb_ref,
                  panc_ref, tcl_ref, ob_refs, o_ref):
    box_l = jnp.zeros((), jnp.float32)
    obj_l = jnp.zeros((), jnp.float32)
    cls_l = jnp.zeros((), jnp.float32)
    for s in range(ns):
        win = g_refs[s]
        r = r_ref[s]
        q = lax.shift_right_logical(r, 4)
        r16 = lax.bitwise_and(r, 15)

        def realign(ch0, width):
            sub = jnp.zeros((width + 15, r.shape[0]), jnp.float32)
            for t in range(8):
                sub = jnp.where(q == t,
                                win[16 * t + ch0:16 * t + ch0 + width + 15, :],
                                sub)
            acc = jnp.zeros((width, r.shape[0]), jnp.float32)
            for t in range(16):
                acc = jnp.where(r16 == t, sub[t:t + width, :], acc)
            return acc

        mf = mf_ref[s]
        cnt = jnp.sum(mf)
        g = realign(0, 5)
        px = 1.0 / (1.0 + jnp.exp(-g[0]))
        py = 1.0 / (1.0 + jnp.exp(-g[1]))
        pw = jnp.minimum(jnp.exp(g[2]), 1000.0) * panc_ref[s, 0]
        ph = jnp.minimum(jnp.exp(g[3]), 1000.0) * panc_ref[s, 1]
        tx = tb_ref[s, 0]
        ty = tb_ref[s, 1]
        tw = tb_ref[s, 2]
        th = tb_ref[s, 3]
        x0 = jnp.maximum(px - pw * 0.5, tx - tw * 0.5)
        y0 = jnp.maximum(py - ph * 0.5, ty - th * 0.5)
        x1 = jnp.minimum(px + pw * 0.5, tx + tw * 0.5)
        y1 = jnp.minimum(py + ph * 0.5, ty + th * 0.5)
        m = ((x0 < x1) & (y0 < y1)).astype(jnp.float32)
        inter = (x1 - x0) * (y1 - y0) * m
        iou = inter / (pw * ph + tw * th - inter)
        box_l += jnp.sum(jnp.where(mf > 0.0, 1.0 - iou, 0.0)) / cnt
        x4 = g[4]
        corr = jnp.sum(jnp.where(first_ref[s] > 0.0,
                                 _softplus(-x4) - _softplus(x4), 0.0))
        obj_l += (ob_refs[s][...][0, 0] + corr) * (1.0 / M_list[s])
        tcl = tcl_ref[s]
        E = mf.shape[0]
        acc = jnp.zeros((E,), jnp.float32)
        CH = 16
        for c0 in range(5, 5 + nc, CH):
            xc = realign(c0, CH)
            spx = _softplus(xc)
            spn = _softplus(-xc)
            hot = (lax.broadcasted_iota(jnp.int32, (CH, E), 0) + (c0 - 5)) == tcl
            acc += jnp.sum(jnp.where(hot, spn, spx), axis=0)
        cls_l += jnp.sum(jnp.where(mf > 0.0, acc, 0.0)) / (cnt * nc)
    total = box_l * _BW + obj_l * _OW + cls_l * _CW
    o_ref[...] = total.reshape(1, 1)


def _run_combine(gTs, mfS, firstS, rS, tbS, pancS, tclS, obs, ns, nc, M_list):
    def body(g0, g1, g2, mf, first, r, tb, panc, tcl, ob0, ob1, ob2, o_ref):
        _combine_body(ns, nc, M_list, (g0, g1, g2), mf, first, r, tb, panc,
                      tcl, (ob0, ob1, ob2), o_ref)

    return pl.pallas_call(
        body,
        out_shape=jax.ShapeDtypeStruct((1, 1), jnp.float32),
    )(*gTs, mfS, firstS, rS, tbS, pancS, tclS, *obs)


def kernel(inf0, inf1, inf2, targets, anchors, strides):
    infs = [inf0, inf1, inf2]
    ns, na, _ = anchors.shape
    B, N, _ = targets.shape
    C = inf0.shape[-1]
    nc = C - 5
    H_list = [f.shape[2] for f in infs]
    E = na * B * N
    NW = 32
    EP = ((E + 8 * NW - 1) // (8 * NW)) * (8 * NW)
    rows_per_worker = EP // NW

    tgtT = jnp.transpose(targets.reshape(B * N, 5)).reshape(5, B, N)
    mf9, first9, grow9, r9, tb, tcl = _run_assign(
        tgtT, anchors, strides, na, ns, H_list)

    pad = EP - E
    K2 = 2
    tables, idxs, obs = [], [], []
    for s in range(ns):
        flat128 = infs[s].reshape(-1, 128)
        ngran = flat128.shape[0]
        grow = jnp.concatenate(
            [grow9[s * na:(s + 1) * na].reshape(E), jnp.zeros((pad,), jnp.int32)])
        idx2 = jnp.minimum(
            grow[None, :] + jnp.arange(K2, dtype=jnp.int32)[:, None],
            ngran - 1).reshape(K2 * EP)
        tables.append(flat128)
        idxs.append(idx2)
        obs.append(_run_objsum(infs[s].reshape(-1, 16 * C), C))
    wins = _sc_gather3(tables, idxs, (K2 * EP) // NW, 128)
    gTs = [w.reshape(K2, EP, 128).transpose(0, 2, 1).reshape(K2 * 128, EP)
           for w in wins]

    def expand(x):
        return jnp.concatenate(
            [jnp.broadcast_to(x[None], (na, B, N)).reshape(E),
         jnp.zeros((pad,), x.dtype)])

    mfS = jnp.stack([jnp.concatenate([mf9[s * na:(s + 1) * na].reshape(E),
                                      jnp.zeros((pad,), jnp.float32)])
                     for s in range(ns)])
    firstS = jnp.stack([jnp.concatenate([first9[s * na:(s + 1) * na].reshape(E),
                                         jnp.zeros((pad,), jnp.float32)])
                        for s in range(ns)])
    rS = jnp.stack([jnp.concatenate([r9[s * na:(s + 1) * na].reshape(E),
                                     jnp.zeros((pad,), jnp.int32)])
                    for s in range(ns)])
    tbS = jnp.stack([jnp.stack([expand(tb[s, k]) for k in range(4)])
                     for s in range(ns)])
    panc = anchors / strides[:, None, None]
    pancS = jnp.stack([
        jnp.stack([jnp.concatenate(
            [jnp.broadcast_to(panc[s, :, k, None], (na, B * N)).reshape(E),
             jnp.zeros((pad,), jnp.float32)]) for k in range(2)])
        for s in range(ns)])
    tclS = jnp.stack([expand(tcl) for _ in range(ns)])
    M_list = [B * na * h * h for h in H_list]

    out = _run_combine(gTs, mfS, firstS, rS, tbS, pancS, tclS, obs,
                       ns, nc, M_list)
    return out[0, 0]


# speedup vs baseline: 1.0467x; 1.0467x over previous
import functools

import jax
import jax.numpy as jnp
from jax import lax
from jax.experimental import pallas as pl
from jax.experimental.pallas import tpu as pltpu
from jax.experimental.pallas import tpu_sc as plsc

_IOU_T = 0.5
_BW, _OW, _CW = 3.54, 64.3, 37.4


def _softplus(x):
    return jnp.maximum(x, 0.0) + jnp.log1p(jnp.exp(-jnp.abs(x)))


def _assign_body(na, ns, H_list, tgt_ref, anch_ref, strd_ref,
                 mf_ref, first_ref, grow_ref, r_ref, tb_ref, tcl_ref):
    B, N = tgt_ref.shape[1], tgt_ref.shape[2]
    x = tgt_ref[0]
    y = tgt_ref[1]
    w = tgt_ref[2]
    h = tgt_ref[3]
    c = tgt_ref[4]
    valid = ~((c == -1.0) & (x == -1.0) & (y == -1.0) & (w == -1.0) & (h == -1.0))
    bidx = lax.broadcasted_iota(jnp.int32, (B, N), 0)
    tcl_ref[...] = c.astype(jnp.int32)
    for s in range(ns):
        inv = 1.0 / strd_ref[s]
        tx = x * inv
        ty = y * inv
        tw = w * inv
        th = h * inv
        cx = jnp.floor(tx)
        cy = jnp.floor(ty)
        fx = tx - cx
        fy = ty - cy
        ci = cx.astype(jnp.int32)
        cj = cy.astype(jnp.int32)
        tb_ref[s, 0] = fx
        tb_ref[s, 1] = fy
        tb_ref[s, 2] = tw
        tb_ref[s, 3] = th
        tx0 = fx - tw * 0.5
        ty0 = fy - th * 0.5
        tx1 = fx + tw * 0.5
        ty1 = fy + th * 0.5
        ta = tw * th
        Hs = H_list[s]
        for a in range(na):
            aw = anch_ref[s, a, 0] * inv
            ah = anch_ref[s, a, 1] * inv
            ax0 = 0.5 - aw * 0.5
            ay0 = 0.5 - ah * 0.5
            ax1 = 0.5 + aw * 0.5
            ay1 = 0.5 + ah * 0.5
            x0 = jnp.maximum(tx0, ax0)
            y0 = jnp.maximum(ty0, ay0)
            x1 = jnp.minimum(tx1, ax1)
            y1 = jnp.minimum(ty1, ay1)
            m = ((x0 < x1) & (y0 < y1)).astype(jnp.float32)
            inter = (x1 - x0) * (y1 - y0) * m
            iou = inter / (ta + aw * ah - inter)
            mf = (iou > _IOU_T) & valid
            row = ((bidx * na + a) * Hs + cj) * Hs + ci
            eq = row[:, :, None] == row[:, None, :]
            mfj = mf[:, None, :]
            ii = lax.broadcasted_iota(jnp.int32, (B, N, N), 1)
            jj = lax.broadcasted_iota(jnp.int32, (B, N, N), 2)
            dup = jnp.any(eq & mfj & (jj < ii), axis=2)
            sa = s * na + a
            mf_ref[sa] = mf.astype(jnp.float32)
            first_ref[sa] = (mf & ~dup).astype(jnp.float32)
            rowm = jnp.where(mf, row, 0)
            elem = rowm * 85
            grow_ref[sa] = lax.shift_right_logical(elem, 5)
            r_ref[sa] = lax.bitwise_and(elem, 31)


def _run_assign(tgtT, anchors, strides, na, ns, H_list):
    B, N = tgtT.shape[1], tgtT.shape[2]
    body = functools.partial(_assign_body, na, ns, H_list)
    return pl.pallas_call(
        body,
        in_specs=[
            pl.BlockSpec(memory_space=pltpu.VMEM),
            pl.BlockSpec(memory_space=pltpu.SMEM),
            pl.BlockSpec(memory_space=pltpu.SMEM),
        ],
        out_specs=[pl.BlockSpec(memory_space=pltpu.VMEM)] * 6,
        out_shape=[
            jax.ShapeDtypeStruct((ns * na, B, N), jnp.float32),
            jax.ShapeDtypeStruct((ns * na, B, N), jnp.float32),
            jax.ShapeDtypeStruct((ns * na, B, N), jnp.int32),
            jax.ShapeDtypeStruct((ns * na, B, N), jnp.int32),
            jax.ShapeDtypeStruct((ns, 4, B, N), jnp.float32),
            jax.ShapeDtypeStruct((B, N), jnp.int32),
        ],
    )(tgtT, anchors, strides)


def _sc_gather3(tables, idxs, rows_per_worker, W):
    NROW = idxs[0].shape[0]
    mesh = plsc.VectorSubcoreMesh(core_axis_name="c", subcore_axis_name="s")
    info = plsc.get_sparse_core_info()
    ncore = info.num_cores
    half = rows_per_worker // 2

    @functools.partial(
        pl.kernel,
        out_type=[jax.ShapeDtypeStruct((NROW, W), jnp.float32)] * 3,
        mesh=mesh,
        compiler_params=pltpu.CompilerParams(use_tc_tiling_on_sc=False),
        scratch_types=[
            [pltpu.VMEM((rows_per_worker,), jnp.int32)] * 3,
            [pltpu.VMEM((rows_per_worker, W), jnp.float32)] * 3,
            pltpu.SemaphoreType.DMA,
        ],
    )
    def gather_k(t0, t1, t2, i0, i1, i2, o0, o1, o2, idx_v, rows_v, sem):
        wid = lax.axis_index("s") * ncore + lax.axis_index("c")
        base = wid * rows_per_worker
        tabs, idx_hbm, outs = (t0, t1, t2), (i0, i1, i2), (o0, o1, o2)
        for s in range(3):
            pltpu.sync_copy(idx_hbm[s].at[pl.ds(base, rows_per_worker)],
                            idx_v[s])
        copies = []
        for s in range(3):
            for h in range(2):
                copies.append(pltpu.async_copy(
                    tabs[s].at[idx_v[s].at[pl.ds(h * half, half)]],
                    rows_v[s].at[pl.ds(h * half, half)], sem))
        for cp in copies:
            cp.wait()
        for s in range(3):
            pltpu.sync_copy(rows_v[s], outs[s].at[pl.ds(base, rows_per_worker)])

    return gather_k(*tables, *idxs)


def _obj_body(C, x_ref, o_ref):
    i = pl.program_id(0)

    @pl.when(i == 0)
    def _():
        o_ref[...] = jnp.zeros_like(o_ref)

    W = 16 * C
    col = lax.broadcasted_iota(jnp.int32, (W, 16), 0)
    jj = lax.broadcasted_iota(jnp.int32, (W, 16), 1)
    sel = (col == C * jj + 4).astype(jnp.float32)
    x4 = jax.lax.dot_general(x_ref[...], sel, (((1,), (0,)), ((), ())),
                             preferred_element_type=jnp.float32)
    o_ref[...] += jnp.sum(_softplus(x4)).reshape(1, 1)


def _run_objsum(flat16rows, C):
    G, W = flat16rows.shape
    block_rows = min(1024, G)
    grid = G // block_rows
    return pl.pallas_call(
        functools.partial(_obj_body, C),
        grid=(grid,),
        in_specs=[pl.BlockSpec((block_rows, W), lambda i: (i, 0))],
        out_specs=pl.BlockSpec((1, 1), lambda i: (0, 0)),
        out_shape=jax.ShapeDtypeStruct((1, 1), jnp.float32),
    )(flat16rows)


def _combine_body(ns, nc, M_list, g_refs, mf_ref, first_ref, r_ref, tb_ref,
                  panc_ref, tcl_ref, ob_refs, o_ref):
    box_l = jnp.zeros((), jnp.float32)
    obj_l = jnp.zeros((), jnp.float32)
    cls_l = jnp.zeros((), jnp.float32)
    for s in range(ns):
        win = g_refs[s]
        r = r_ref[s]
        q = lax.shift_right_logical(r, 4)
        r16 = lax.bitwise_and(r, 15)

        def realign(ch0, width):
            sub = jnp.zeros((width + 15, r.shape[0]), jnp.float32)
            for t in range(2):
                sub = jnp.where(q == t,
                                win[16 * t + ch0:16 * t + ch0 + width + 15, :],
                                sub)
            acc = jnp.zeros((width, r.shape[0]), jnp.float32)
            for t in range(16):
                acc = jnp.where(r16 == t, sub[t:t + width, :], acc)
            return acc

        mf = mf_ref[s]
        cnt = jnp.sum(mf)
        g = realign(0, 5)
        px = 1.0 / (1.0 + jnp.exp(-g[0]))
        py = 1.0 / (1.0 + jnp.exp(-g[1]))
        pw = jnp.minimum(jnp.exp(g[2]), 1000.0) * panc_ref[s, 0]
        ph = jnp.minimum(jnp.exp(g[3]), 1000.0) * panc_ref[s, 1]
        tx = tb_ref[s, 0]
        ty = tb_ref[s, 1]
        tw = tb_ref[s, 2]
        th = tb_ref[s, 3]
        x0 = jnp.maximum(px - pw * 0.5, tx - tw * 0.5)
        y0 = jnp.maximum(py - ph * 0.5, ty - th * 0.5)
        x1 = jnp.minimum(px + pw * 0.5, tx + tw * 0.5)
        y1 = jnp.minimum(py + ph * 0.5, ty + th * 0.5)
        m = ((x0 < x1) & (y0 < y1)).astype(jnp.float32)
        inter = (x1 - x0) * (y1 - y0) * m
        iou = inter / (pw * ph + tw * th - inter)
        box_l += jnp.sum(jnp.where(mf > 0.0, 1.0 - iou, 0.0)) / cnt
        x4 = g[4]
        corr = jnp.sum(jnp.where(first_ref[s] > 0.0,
                                 _softplus(-x4) - _softplus(x4), 0.0))
        obj_l += (ob_refs[s][...][0, 0] + corr) * (1.0 / M_list[s])
        tcl = tcl_ref[s]
        E = mf.shape[0]
        acc = jnp.zeros((E,), jnp.float32)
        CH = 16
        for c0 in range(5, 5 + nc, CH):
            xc = realign(c0, CH)
            spx = _softplus(xc)
            spn = _softplus(-xc)
            hot = (lax.broadcasted_iota(jnp.int32, (CH, E), 0) + (c0 - 5)) == tcl
            acc += jnp.sum(jnp.where(hot, spn, spx), axis=0)
        cls_l += jnp.sum(jnp.where(mf > 0.0, acc, 0.0)) / (cnt * nc)
    total = box_l * _BW + obj_l * _OW + cls_l * _CW
    o_ref[...] = total.reshape(1, 1)


def _run_combine(gTs, mfS, firstS, rS, tbS, pancS, tclS, obs, ns, nc, M_list):
    def body(g0, g1, g2, mf, first, r, tb, panc, tcl, ob0, ob1, ob2, o_ref):
        _combine_body(ns, nc, M_list, (g0, g1, g2), mf, first, r, tb, panc,
                      tcl, (ob0, ob1, ob2), o_ref)

    return pl.pallas_call(
        body,
        out_shape=jax.ShapeDtypeStruct((1, 1), jnp.float32),
    )(*gTs, mfS, firstS, rS, tbS, pancS, tclS, *obs)


def kernel(inf0, inf1, inf2, targets, anchors, strides):
    infs = [inf0, inf1, inf2]
    ns, na, _ = anchors.shape
    B, N, _ = targets.shape
    C = inf0.shape[-1]
    nc = C - 5
    H_list = [f.shape[2] for f in infs]
    E = na * B * N
    NW = 32
    EP = ((E + 8 * NW - 1) // (8 * NW)) * (8 * NW)
    rows_per_worker = EP // NW

    tgtT = jnp.transpose(targets.reshape(B * N, 5)).reshape(5, B, N)
    mf9, first9, grow9, r9, tb, tcl = _run_assign(
        tgtT, anchors, strides, na, ns, H_list)

    pad = EP - E
    K4 = 4
    tables, idxs = [], []
    for s in range(ns):
        flat32 = infs[s].reshape(-1, 32)
        ngran = flat32.shape[0]
        grow = jnp.concatenate(
            [grow9[s * na:(s + 1) * na].reshape(E), jnp.zeros((pad,), jnp.int32)])
        idx4 = jnp.minimum(
            grow[None, :] + jnp.arange(K4, dtype=jnp.int32)[:, None],
            ngran - 1).reshape(K4 * EP)
        tables.append(flat32)
        idxs.append(idx4)
    wins = _sc_gather3(tables, idxs, (K4 * EP) // NW, 32)
    obs = [_run_objsum(infs[s].reshape(-1, 16 * C), C) for s in range(ns)]
    gTs = [w.reshape(K4, EP, 32).transpose(0, 2, 1).reshape(K4 * 32, EP)
           for w in wins]

    def expand(x):
        return jnp.concatenate(
            [jnp.broadcast_to(x[None], (na, B, N)).reshape(E),
         jnp.zeros((pad,), x.dtype)])

    mfS = jnp.stack([jnp.concatenate([mf9[s * na:(s + 1) * na].reshape(E),
                                      jnp.zeros((pad,), jnp.float32)])
                     for s in range(ns)])
    firstS = jnp.stack([jnp.concatenate([first9[s * na:(s + 1) * na].reshape(E),
                                         jnp.zeros((pad,), jnp.float32)])
                        for s in range(ns)])
    rS = jnp.stack([jnp.concatenate([r9[s * na:(s + 1) * na].reshape(E),
                                     jnp.zeros((pad,), jnp.int32)])
                    for s in range(ns)])
    tbS = jnp.stack([jnp.stack([expand(tb[s, k]) for k in range(4)])
                     for s in range(ns)])
    panc = anchors / strides[:, None, None]
    pancS = jnp.stack([
        jnp.stack([jnp.concatenate(
            [jnp.broadcast_to(panc[s, :, k, None], (na, B * N)).reshape(E),
             jnp.zeros((pad,), jnp.float32)]) for k in range(2)])
        for s in range(ns)])
    tclS = jnp.stack([expand(tcl) for _ in range(ns)])
    M_list = [B * na * h * h for h in H_list]

    out = _run_combine(gTs, mfS, firstS, rS, tbS, pancS, tclS, obs,
                       ns, nc, M_list)
    return out[0, 0]


# speedup vs baseline: 1.3187x; 1.2599x over previous
import functools

import jax
import jax.numpy as jnp
from jax import lax
from jax.experimental import pallas as pl
from jax.experimental.pallas import tpu as pltpu
from jax.experimental.pallas import tpu_sc as plsc

_IOU_T = 0.5
_BW, _OW, _CW = 3.54, 64.3, 37.4


def _softplus(x):
    return jnp.maximum(x, 0.0) + jnp.log1p(jnp.exp(-jnp.abs(x)))


def _assign_body(na, ns, H_list, tgt_ref, anch_ref, strd_ref,
                 mf_ref, first_ref, tb_ref, tcl_ref):
    B, N = tgt_ref.shape[1], tgt_ref.shape[2]
    x = tgt_ref[0]
    y = tgt_ref[1]
    w = tgt_ref[2]
    h = tgt_ref[3]
    c = tgt_ref[4]
    valid = ~((c == -1.0) & (x == -1.0) & (y == -1.0) & (w == -1.0) & (h == -1.0))
    bidx = lax.broadcasted_iota(jnp.int32, (B, N), 0)
    tcl_ref[...] = c.astype(jnp.int32)
    for s in range(ns):
        inv = 1.0 / strd_ref[s]
        tx = x * inv
        ty = y * inv
        tw = w * inv
        th = h * inv
        cx = jnp.floor(tx)
        cy = jnp.floor(ty)
        fx = tx - cx
        fy = ty - cy
        ci = cx.astype(jnp.int32)
        cj = cy.astype(jnp.int32)
        tb_ref[s, 0] = fx
        tb_ref[s, 1] = fy
        tb_ref[s, 2] = tw
        tb_ref[s, 3] = th
        tx0 = fx - tw * 0.5
        ty0 = fy - th * 0.5
        tx1 = fx + tw * 0.5
        ty1 = fy + th * 0.5
        ta = tw * th
        Hs = H_list[s]
        for a in range(na):
            aw = anch_ref[s, a, 0] * inv
            ah = anch_ref[s, a, 1] * inv
            ax0 = 0.5 - aw * 0.5
            ay0 = 0.5 - ah * 0.5
            ax1 = 0.5 + aw * 0.5
            ay1 = 0.5 + ah * 0.5
            x0 = jnp.maximum(tx0, ax0)
            y0 = jnp.maximum(ty0, ay0)
            x1 = jnp.minimum(tx1, ax1)
            y1 = jnp.minimum(ty1, ay1)
            m = ((x0 < x1) & (y0 < y1)).astype(jnp.float32)
            inter = (x1 - x0) * (y1 - y0) * m
            iou = inter / (ta + aw * ah - inter)
            mf = (iou > _IOU_T) & valid
            row = ((bidx * na + a) * Hs + cj) * Hs + ci
            eq = row[:, :, None] == row[:, None, :]
            mfj = mf[:, None, :]
            ii = lax.broadcasted_iota(jnp.int32, (B, N, N), 1)
            jj = lax.broadcasted_iota(jnp.int32, (B, N, N), 2)
            dup = jnp.any(eq & mfj & (jj < ii), axis=2)
            sa = s * na + a
            mf_ref[sa] = mf.astype(jnp.float32)
            first_ref[sa] = (mf & ~dup).astype(jnp.float32)


def _run_assign(tgtT, anchors, strides, na, ns, H_list):
    B, N = tgtT.shape[1], tgtT.shape[2]
    body = functools.partial(_assign_body, na, ns, H_list)
    return pl.pallas_call(
        body,
        in_specs=[
            pl.BlockSpec(memory_space=pltpu.VMEM),
            pl.BlockSpec(memory_space=pltpu.SMEM),
            pl.BlockSpec(memory_space=pltpu.SMEM),
        ],
        out_specs=[pl.BlockSpec(memory_space=pltpu.VMEM)] * 4,
        out_shape=[
            jax.ShapeDtypeStruct((ns * na, B, N), jnp.float32),
            jax.ShapeDtypeStruct((ns * na, B, N), jnp.float32),
            jax.ShapeDtypeStruct((ns, 4, B, N), jnp.float32),
            jax.ShapeDtypeStruct((B, N), jnp.int32),
        ],
    )(tgtT, anchors, strides)


def _sc_gather3(tables, idxs, rows_per_worker, W):
    NROW = idxs[0].shape[0]
    mesh = plsc.VectorSubcoreMesh(core_axis_name="c", subcore_axis_name="s")
    info = plsc.get_sparse_core_info()
    ncore = info.num_cores
    half = rows_per_worker // 2

    @functools.partial(
        pl.kernel,
        out_type=[jax.ShapeDtypeStruct((NROW, W), jnp.float32)] * 3,
        mesh=mesh,
        compiler_params=pltpu.CompilerParams(use_tc_tiling_on_sc=False),
        scratch_types=[
            [pltpu.VMEM((rows_per_worker,), jnp.int32)] * 3,
            [pltpu.VMEM((rows_per_worker, W), jnp.float32)] * 3,
            pltpu.SemaphoreType.DMA,
        ],
    )
    def gather_k(t0, t1, t2, i0, i1, i2, o0, o1, o2, idx_v, rows_v, sem):
        wid = lax.axis_index("s") * ncore + lax.axis_index("c")
        base = wid * rows_per_worker
        tabs, idx_hbm, outs = (t0, t1, t2), (i0, i1, i2), (o0, o1, o2)
        for s in range(3):
            pltpu.sync_copy(idx_hbm[s].at[pl.ds(base, rows_per_worker)],
                            idx_v[s])
        copies = []
        for s in range(3):
            for h in range(2):
                copies.append(pltpu.async_copy(
                    tabs[s].at[idx_v[s].at[pl.ds(h * half, half)]],
                    rows_v[s].at[pl.ds(h * half, half)], sem))
        for cp in copies:
            cp.wait()
        for s in range(3):
            pltpu.sync_copy(rows_v[s], outs[s].at[pl.ds(base, rows_per_worker)])

    return gather_k(*tables, *idxs)


def _obj_body(C, x_ref, o_ref):
    i = pl.program_id(0)

    @pl.when(i == 0)
    def _():
        o_ref[...] = jnp.zeros_like(o_ref)

    W = 16 * C
    col = lax.broadcasted_iota(jnp.int32, (W, 16), 0)
    jj = lax.broadcasted_iota(jnp.int32, (W, 16), 1)
    sel = (col == C * jj + 4).astype(jnp.float32)
    x4 = jax.lax.dot_general(x_ref[...], sel, (((1,), (0,)), ((), ())),
                             preferred_element_type=jnp.float32)
    o_ref[...] += jnp.sum(_softplus(x4)).reshape(1, 1)


def _run_objsum(flat16rows, C):
    G, W = flat16rows.shape
    block_rows = min(1024, G)
    grid = G // block_rows
    return pl.pallas_call(
        functools.partial(_obj_body, C),
        grid=(grid,),
        in_specs=[pl.BlockSpec((block_rows, W), lambda i: (i, 0))],
        out_specs=pl.BlockSpec((1, 1), lambda i: (0, 0)),
        out_shape=jax.ShapeDtypeStruct((1, 1), jnp.float32),
    )(flat16rows)


def _combine_body(ns, nc, M_list, g_refs, mf_ref, first_ref, r_ref, tb_ref,
                  panc_ref, tcl_ref, ob_refs, o_ref):
    box_l = jnp.zeros((), jnp.float32)
    obj_l = jnp.zeros((), jnp.float32)
    cls_l = jnp.zeros((), jnp.float32)
    for s in range(ns):
        win = g_refs[s]
        r = r_ref[s]
        q = lax.shift_right_logical(r, 4)
        r16 = lax.bitwise_and(r, 15)

        def realign(ch0, width):
            sub = jnp.zeros((width + 15, r.shape[0]), jnp.float32)
            for t in range(2):
                sub = jnp.where(q == t,
                                win[16 * t + ch0:16 * t + ch0 + width + 15, :],
                                sub)
            acc = jnp.zeros((width, r.shape[0]), jnp.float32)
            for t in range(16):
                acc = jnp.where(r16 == t, sub[t:t + width, :], acc)
            return acc

        mf = mf_ref[s]
        cnt = jnp.sum(mf)
        g = realign(0, 5)
        px = 1.0 / (1.0 + jnp.exp(-g[0]))
        py = 1.0 / (1.0 + jnp.exp(-g[1]))
        pw = jnp.minimum(jnp.exp(g[2]), 1000.0) * panc_ref[s, 0]
        ph = jnp.minimum(jnp.exp(g[3]), 1000.0) * panc_ref[s, 1]
        tx = tb_ref[s, 0]
        ty = tb_ref[s, 1]
        tw = tb_ref[s, 2]
        th = tb_ref[s, 3]
        x0 = jnp.maximum(px - pw * 0.5, tx - tw * 0.5)
        y0 = jnp.maximum(py - ph * 0.5, ty - th * 0.5)
        x1 = jnp.minimum(px + pw * 0.5, tx + tw * 0.5)
        y1 = jnp.minimum(py + ph * 0.5, ty + th * 0.5)
        m = ((x0 < x1) & (y0 < y1)).astype(jnp.float32)
        inter = (x1 - x0) * (y1 - y0) * m
        iou = inter / (pw * ph + tw * th - inter)
        box_l += jnp.sum(jnp.where(mf > 0.0, 1.0 - iou, 0.0)) / cnt
        x4 = g[4]
        corr = jnp.sum(jnp.where(first_ref[s] > 0.0,
                                 _softplus(-x4) - _softplus(x4), 0.0))
        obj_l += (ob_refs[s][...][0, 0] + corr) * (1.0 / M_list[s])
        tcl = tcl_ref[s]
        E = mf.shape[0]
        acc = jnp.zeros((E,), jnp.float32)
        CH = 16
        for c0 in range(5, 5 + nc, CH):
            xc = realign(c0, CH)
            spx = _softplus(xc)
            spn = _softplus(-xc)
            hot = (lax.broadcasted_iota(jnp.int32, (CH, E), 0) + (c0 - 5)) == tcl
            acc += jnp.sum(jnp.where(hot, spn, spx), axis=0)
        cls_l += jnp.sum(jnp.where(mf > 0.0, acc, 0.0)) / (cnt * nc)
    total = box_l * _BW + obj_l * _OW + cls_l * _CW
    o_ref[...] = total.reshape(1, 1)


def _run_combine(gTs, mfS, firstS, rS, tbS, pancS, tclS, obs, ns, nc, M_list):
    def body(g0, g1, g2, mf, first, r, tb, panc, tcl, ob0, ob1, ob2, o_ref):
        _combine_body(ns, nc, M_list, (g0, g1, g2), mf, first, r, tb, panc,
                      tcl, (ob0, ob1, ob2), o_ref)

    return pl.pallas_call(
        body,
        out_shape=jax.ShapeDtypeStruct((1, 1), jnp.float32),
    )(*gTs, mfS, firstS, rS, tbS, pancS, tclS, *obs)


def kernel(inf0, inf1, inf2, targets, anchors, strides):
    infs = [inf0, inf1, inf2]
    ns, na, _ = anchors.shape
    B, N, _ = targets.shape
    C = inf0.shape[-1]
    nc = C - 5
    H_list = [f.shape[2] for f in infs]
    E = na * B * N
    NW = 32
    EP = ((E + 8 * NW - 1) // (8 * NW)) * (8 * NW)
    rows_per_worker = EP // NW

    pad = EP - E
    K4 = 4
    aidx = jnp.arange(na, dtype=jnp.int32)[:, None, None]
    bidx = jnp.arange(B, dtype=jnp.int32)[None, :, None]
    tables, idxs, r_list = [], [], []
    for s in range(ns):
        flat32 = infs[s].reshape(-1, 32)
        ngran = flat32.shape[0]
        Hs = H_list[s]
        inv = 1.0 / strides[s]
        ci = jnp.floor(targets[..., 0] * inv).astype(jnp.int32)[None]
        cj = jnp.floor(targets[..., 1] * inv).astype(jnp.int32)[None]
        elem = (((bidx * na + aidx) * Hs + cj) * Hs + ci) * C
        grow = lax.shift_right_logical(
            jnp.clip(elem, 0, (ngran - 1) * 32), 5).reshape(E)
        r_list.append(jnp.concatenate(
            [lax.bitwise_and(jnp.clip(elem, 0, None), 31).reshape(E),
             jnp.zeros((pad,), jnp.int32)]))
        grow = jnp.concatenate([grow, jnp.zeros((pad,), jnp.int32)])
        idx4 = jnp.minimum(
            grow[None, :] + jnp.arange(K4, dtype=jnp.int32)[:, None],
            ngran - 1).reshape(K4 * EP)
        tables.append(flat32)
        idxs.append(idx4)
    wins = _sc_gather3(tables, idxs, (K4 * EP) // NW, 32)

    tgtT = jnp.transpose(targets.reshape(B * N, 5)).reshape(5, B, N)
    mf9, first9, tb, tcl = _run_assign(tgtT, anchors, strides, na, ns, H_list)
    obs = [_run_objsum(infs[s].reshape(-1, 16 * C), C) for s in range(ns)]
    gTs = [w.reshape(K4, EP, 32).transpose(0, 2, 1).reshape(K4 * 32, EP)
           for w in wins]

    def expand(x):
        return jnp.concatenate(
            [jnp.broadcast_to(x[None], (na, B, N)).reshape(E),
         jnp.zeros((pad,), x.dtype)])

    mfS = jnp.stack([jnp.concatenate([mf9[s * na:(s + 1) * na].reshape(E),
                                      jnp.zeros((pad,), jnp.float32)])
                     for s in range(ns)])
    firstS = jnp.stack([jnp.concatenate([first9[s * na:(s + 1) * na].reshape(E),
                                         jnp.zeros((pad,), jnp.float32)])
                        for s in range(ns)])
    rS = jnp.stack(r_list)
    tbS = jnp.stack([jnp.stack([expand(tb[s, k]) for k in range(4)])
                     for s in range(ns)])
    panc = anchors / strides[:, None, None]
    pancS = jnp.stack([
        jnp.stack([jnp.concatenate(
            [jnp.broadcast_to(panc[s, :, k, None], (na, B * N)).reshape(E),
             jnp.zeros((pad,), jnp.float32)]) for k in range(2)])
        for s in range(ns)])
    tclS = jnp.stack([expand(tcl) for _ in range(ns)])
    M_list = [B * na * h * h for h in H_list]

    out = _run_combine(gTs, mfS, firstS, rS, tbS, pancS, tclS, obs,
                       ns, nc, M_list)
    return out[0, 0]
